# initial kernel scaffold (unmeasured)
import jax
import jax.numpy as jnp
from jax import lax
from jax.experimental import pallas as pl
from jax.experimental.pallas import tpu as pltpu

B = 16
H = 16
D = 64
BS = 16
NP_LOC = 128
NB = 128
NKEY = NP_LOC * BS
SCALE = D ** -0.5


def kernel(Q, K, V, bt, lens):
    lens2d = lens.reshape(B, 1)

    def body(q_ref, k_ref, v_ref, bt_ref, lens_ref, out_ref,
             comm_ref, send_sem, recv_sem):
        my_x = lax.axis_index("x")
        my_y = lax.axis_index("y")
        my_z = lax.axis_index("z")

        bt3 = bt_ref[:, :][:, :, None]
        pids = lax.broadcasted_iota(jnp.int32, (B, NB, NP_LOC), 2) \
            + my_x * NP_LOC
        jidx = lax.broadcasted_iota(jnp.int32, (B, NB, 1), 1)
        valid = jidx < lens_ref[:, :][:, None, :]
        hit = jnp.logical_and(bt3 == pids, valid)
        cnt = jnp.sum(jnp.where(hit, 1.0, 0.0), axis=1)

        e_rows = lax.broadcasted_iota(jnp.int32, (NP_LOC, NKEY), 0)
        e_cols = lax.broadcasted_iota(jnp.int32, (NP_LOC, NKEY), 1) // BS
        expand = jnp.where(e_rows == e_cols, 1.0, 0.0).astype(jnp.bfloat16)
        c_key = lax.dot_general(
            cnt.astype(jnp.bfloat16), expand,
            (((1,), (0,)), ((), ())),
            preferred_element_type=jnp.float32,
        )
        key_on = c_key > 0.0

        q_all = q_ref[:, 0, :, :].astype(jnp.bfloat16)
        for h in range(H):
            q_h = q_all[:, h, :]
            k_h = k_ref[:, :, h, :].reshape(NKEY, D).astype(jnp.bfloat16)
            v_h = v_ref[:, :, h, :].reshape(NKEY, D).astype(jnp.bfloat16)
            s = lax.dot_general(
                q_h, k_h, (((1,), (1,)), ((), ())),
                preferred_element_type=jnp.float32,
            ) * SCALE
            s = jnp.where(key_on, s, -1e30)
            m_h = jnp.max(s, axis=1, keepdims=True)
            p = c_key * jnp.exp(s - m_h)
            l_h = jnp.sum(p, axis=1, keepdims=True)
            acc_h = lax.dot_general(
                p.astype(jnp.bfloat16), v_h, (((1,), (0,)), ((), ())),
                preferred_element_type=jnp.float32,
            )
            comm_ref[0, :, h, 0:D] = acc_h
            comm_ref[0, :, h, D:D + 1] = m_h
            comm_ref[0, :, h, D + 1:D + 2] = l_h

        rdma = pltpu.make_async_remote_copy(
            src_ref=comm_ref.at[0],
            dst_ref=comm_ref.at[1],
            send_sem=send_sem,
            recv_sem=recv_sem,
            device_id=(1 - my_x, my_y, my_z),
            device_id_type=pl.DeviceIdType.MESH,
        )
        rdma.start()
        rdma.wait()

        acc1 = comm_ref[0, :, :, 0:D]
        m1 = comm_ref[0, :, :, D:D + 1]
        l1 = comm_ref[0, :, :, D + 1:D + 2]
        acc2 = comm_ref[1, :, :, 0:D]
        m2 = comm_ref[1, :, :, D:D + 1]
        l2 = comm_ref[1, :, :, D + 1:D + 2]
        m_new = jnp.maximum(m1, m2)
        a1 = jnp.exp(m1 - m_new)
        a2 = jnp.exp(m2 - m_new)
        l_tot = l1 * a1 + l2 * a2
        out_ref[:, 0, :, :] = (acc1 * a1 + acc2 * a2) / l_tot

    return pl.pallas_call(
        body,
        out_shape=jax.ShapeDtypeStruct((B, 1, H, D), jnp.float32),
        in_specs=[
            pl.BlockSpec(memory_space=pltpu.VMEM),
            pl.BlockSpec(memory_space=pltpu.VMEM),
            pl.BlockSpec(memory_space=pltpu.VMEM),
            pl.BlockSpec(memory_space=pltpu.VMEM),
            pl.BlockSpec(memory_space=pltpu.VMEM),
        ],
        out_specs=pl.BlockSpec(memory_space=pltpu.VMEM),
        scratch_shapes=[
            pltpu.VMEM((2, B, H, 128), jnp.float32),
            pltpu.SemaphoreType.DMA,
            pltpu.SemaphoreType.DMA,
        ],
        compiler_params=pltpu.CompilerParams(collective_id=0),
    )(Q, K, V, bt, lens2d)


# baseline (device time: 84427 ns/iter reference)
import jax
import jax.numpy as jnp
from jax import lax
from jax.experimental import pallas as pl
from jax.experimental.pallas import tpu as pltpu

B = 16
H = 16
D = 64
BS = 16
NP_LOC = 128
NB = 128
NKEY = NP_LOC * BS
SCALE = D ** -0.5


def kernel(Q, K, V, bt, lens):
    lens2d = lens.reshape(B, 1)

    def body(q_ref, k_ref, v_ref, bt_ref, lens_ref, out_ref,
             comm_ref, send_sem, recv_sem):
        my_x = lax.axis_index("x")
        my_y = lax.axis_index("y")
        my_z = lax.axis_index("z")

        bt3 = bt_ref[:, :][:, :, None]
        pids = lax.broadcasted_iota(jnp.int32, (B, NB, NP_LOC), 2) \
            + my_x * NP_LOC
        jidx = lax.broadcasted_iota(jnp.int32, (B, NB, 1), 1)
        valid = jidx < lens_ref[:, :][:, None, :]
        hit = jnp.logical_and(bt3 == pids, valid)
        cnt = jnp.sum(jnp.where(hit, 1.0, 0.0), axis=1)

        e_rows = lax.broadcasted_iota(jnp.int32, (NP_LOC, NKEY), 0)
        e_cols = lax.broadcasted_iota(jnp.int32, (NP_LOC, NKEY), 1) // BS
        expand = jnp.where(e_rows == e_cols, 1.0, 0.0).astype(jnp.bfloat16)
        c_key = lax.dot_general(
            cnt.astype(jnp.bfloat16), expand,
            (((1,), (0,)), ((), ())),
            preferred_element_type=jnp.float32,
        )
        key_on = c_key > 0.0

        q_all = q_ref[:, 0, :, :].astype(jnp.bfloat16)
        for h in range(H):
            q_h = q_all[:, h, :]
            k_h = k_ref[:, :, h, :].reshape(NKEY, D).astype(jnp.bfloat16)
            v_h = v_ref[:, :, h, :].reshape(NKEY, D).astype(jnp.bfloat16)
            s = lax.dot_general(
                q_h, k_h, (((1,), (1,)), ((), ())),
                preferred_element_type=jnp.float32,
            ) * SCALE
            s = jnp.where(key_on, s, -1e30)
            m_h = jnp.max(s, axis=1, keepdims=True)
            p = c_key * jnp.exp(s - m_h)
            l_h = jnp.sum(p, axis=1, keepdims=True)
            acc_h = lax.dot_general(
                p.astype(jnp.bfloat16), v_h, (((1,), (0,)), ((), ())),
                preferred_element_type=jnp.float32,
            )
            comm_ref[0, :, h, 0:D] = acc_h
            comm_ref[0, :, h, D:D + 1] = m_h
            comm_ref[0, :, h, D + 1:D + 2] = l_h

        rdma = pltpu.make_async_remote_copy(
            src_ref=comm_ref.at[0],
            dst_ref=comm_ref.at[1],
            send_sem=send_sem,
            recv_sem=recv_sem,
            device_id=(1 - my_x, my_y, my_z),
            device_id_type=pl.DeviceIdType.MESH,
        )
        rdma.start()
        rdma.wait()

        acc1 = comm_ref[0, :, :, 0:D]
        m1 = comm_ref[0, :, :, D:D + 1]
        l1 = comm_ref[0, :, :, D + 1:D + 2]
        acc2 = comm_ref[1, :, :, 0:D]
        m2 = comm_ref[1, :, :, D:D + 1]
        l2 = comm_ref[1, :, :, D + 1:D + 2]
        m_new = jnp.maximum(m1, m2)
        a1 = jnp.exp(m1 - m_new)
        a2 = jnp.exp(m2 - m_new)
        l_tot = l1 * a1 + l2 * a2
        out_ref[:, 0, :, :] = (acc1 * a1 + acc2 * a2) / l_tot

    return pl.pallas_call(
        body,
        out_shape=jax.ShapeDtypeStruct((B, 1, H, D), jnp.float32),
        in_specs=[
            pl.BlockSpec(memory_space=pltpu.VMEM),
            pl.BlockSpec(memory_space=pltpu.VMEM),
            pl.BlockSpec(memory_space=pltpu.VMEM),
            pl.BlockSpec(memory_space=pltpu.VMEM),
            pl.BlockSpec(memory_space=pltpu.VMEM),
        ],
        out_specs=pl.BlockSpec(memory_space=pltpu.VMEM),
        scratch_shapes=[
            pltpu.VMEM((2, B, H, 128), jnp.float32),
            pltpu.SemaphoreType.DMA,
            pltpu.SemaphoreType.DMA,
        ],
    )(Q, K, V, bt, lens2d)


# device time: 34394 ns/iter; 2.4547x vs baseline; 2.4547x over previous
import jax
import jax.numpy as jnp
from jax import lax
from jax.experimental import pallas as pl
from jax.experimental.pallas import tpu as pltpu

B = 16
H = 16
D = 64
BS = 16
NP_LOC = 128
NB = 128
NKEY = NP_LOC * BS
SCALE = D ** -0.5


def kernel(Q, K, V, bt, lens):
    lens2d = lens.reshape(B, 1)
    q_t = jnp.transpose(Q[:, 0, :, :], (1, 0, 2)).astype(jnp.bfloat16)
    k_t = jnp.transpose(K.reshape(NKEY, H, D), (1, 0, 2)).astype(jnp.bfloat16)
    v_t = jnp.transpose(V.reshape(NKEY, H, D), (1, 0, 2)).astype(jnp.bfloat16)

    def body(q_ref, k_ref, v_ref, bt_ref, lens_ref, out_ref,
             comm_ref, send_sem, recv_sem):
        my_x = lax.axis_index("x")
        my_y = lax.axis_index("y")
        my_z = lax.axis_index("z")

        bt3 = bt_ref[:, :][:, :, None]
        pids = lax.broadcasted_iota(jnp.int32, (B, NB, NP_LOC), 2) \
            + my_x * NP_LOC
        jidx = lax.broadcasted_iota(jnp.int32, (B, NB, 1), 1)
        valid = jidx < lens_ref[:, :][:, None, :]
        hit = jnp.logical_and(bt3 == pids, valid)
        cnt = jnp.sum(jnp.where(hit, 1.0, 0.0), axis=1)

        e_rows = lax.broadcasted_iota(jnp.int32, (NP_LOC, NKEY), 0)
        e_cols = lax.broadcasted_iota(jnp.int32, (NP_LOC, NKEY), 1) // BS
        expand = jnp.where(e_rows == e_cols, 1.0, 0.0).astype(jnp.bfloat16)
        c_key = lax.dot_general(
            cnt.astype(jnp.bfloat16), expand,
            (((1,), (0,)), ((), ())),
            preferred_element_type=jnp.float32,
        )
        key_on = (c_key > 0.0)[None, :, :]

        s = lax.dot_general(
            q_ref[:, :, :], k_ref[:, :, :],
            (((2,), (2,)), ((0,), (0,))),
            preferred_element_type=jnp.float32,
        ) * SCALE
        s = jnp.where(key_on, s, -1e30)
        m = jnp.max(s, axis=2, keepdims=True)
        p = c_key[None, :, :] * jnp.exp(s - m)
        l = jnp.sum(p, axis=2, keepdims=True)
        acc = lax.dot_general(
            p.astype(jnp.bfloat16), v_ref[:, :, :],
            (((2,), (1,)), ((0,), (0,))),
            preferred_element_type=jnp.float32,
        )

        comm_ref[0, :, :, 0:D] = jnp.transpose(acc, (1, 0, 2))
        comm_ref[0, :, :, D:D + 1] = jnp.transpose(m, (1, 0, 2))
        comm_ref[0, :, :, D + 1:D + 2] = jnp.transpose(l, (1, 0, 2))

        rdma = pltpu.make_async_remote_copy(
            src_ref=comm_ref.at[0],
            dst_ref=comm_ref.at[1],
            send_sem=send_sem,
            recv_sem=recv_sem,
            device_id=(1 - my_x, my_y, my_z),
            device_id_type=pl.DeviceIdType.MESH,
        )
        rdma.start()
        rdma.wait()

        acc1 = comm_ref[0, :, :, 0:D]
        m1 = comm_ref[0, :, :, D:D + 1]
        l1 = comm_ref[0, :, :, D + 1:D + 2]
        acc2 = comm_ref[1, :, :, 0:D]
        m2 = comm_ref[1, :, :, D:D + 1]
        l2 = comm_ref[1, :, :, D + 1:D + 2]
        m_new = jnp.maximum(m1, m2)
        a1 = jnp.exp(m1 - m_new)
        a2 = jnp.exp(m2 - m_new)
        l_tot = l1 * a1 + l2 * a2
        out_ref[:, 0, :, :] = (acc1 * a1 + acc2 * a2) / l_tot

    return pl.pallas_call(
        body,
        out_shape=jax.ShapeDtypeStruct((B, 1, H, D), jnp.float32),
        in_specs=[
            pl.BlockSpec(memory_space=pltpu.VMEM),
            pl.BlockSpec(memory_space=pltpu.VMEM),
            pl.BlockSpec(memory_space=pltpu.VMEM),
            pl.BlockSpec(memory_space=pltpu.VMEM),
            pl.BlockSpec(memory_space=pltpu.VMEM),
        ],
        out_specs=pl.BlockSpec(memory_space=pltpu.VMEM),
        scratch_shapes=[
            pltpu.VMEM((2, B, H, 128), jnp.float32),
            pltpu.SemaphoreType.DMA,
            pltpu.SemaphoreType.DMA,
        ],
    )(q_t, k_t, v_t, bt, lens2d)
